# baseline (speedup 1.0000x reference)
import functools

import jax
import jax.numpy as jnp
from jax import lax
from jax.experimental import pallas as pl
from jax.experimental.pallas import tpu as pltpu
from jax.experimental.pallas import tpu_sc as plsc

V, H, W = 8, 512, 512
N = H * W
NT = V * N
CH = 16384
NC_H = N // CH
C = 8192


def _proj_body(T_ref, K_ref, h_ref, cls_ref, conf_ref, maskf_ref, dep_ref,
               lin_ref, dm_ref, tab_ref):
    v = pl.program_id(0)
    c = pl.program_id(1)

    hT = h_ref[0]
    T = T_ref[0]
    wh = jax.lax.dot_general(T, hT, (((1,), (0,)), ((), ())),
                             preferred_element_type=jnp.float32)
    pc = wh[0:3] / wh[3:4]
    img = jax.lax.dot_general(K_ref[...], pc, (((1,), (0,)), ((), ())),
                              preferred_element_type=jnp.float32)
    uv = img[0:2] / img[2:3]
    ij = uv.astype(jnp.int32)
    ix = ij[0:1]
    iy = ij[1:2]
    valid = (ix >= 0) & (ix < W) & (iy >= 0) & (iy < H)
    ixc = jnp.clip(ix, 0, W - 1)
    iyc = jnp.clip(iy, 0, H - 1)

    keep = valid & (cls_ref[0, 0] > 0.0) & (conf_ref[0, 0] > 0.5)
    spread = jax.lax.broadcasted_iota(jnp.int32, (1, CH), 1) & (N - 1)
    lin_ref[0, 0] = jnp.where(keep, iyc * W + ixc, spread)
    dm_ref[0, 0] = jnp.where(keep, pc[2:3], -jnp.inf)

    @pl.when((v == 0) & (c == 0))
    def _():
        tab_ref[...] = jnp.where(maskf_ref[...] > 0.0, dep_ref[...], jnp.inf)


def _project(Ts, K, hmat, cls4, conf4, maskf, depths):
    full = lambda: pl.BlockSpec((H, W), lambda v, c: (0, 0))
    pix = lambda: pl.BlockSpec((1, 1, 1, CH), lambda v, c: (v, c, 0, 0))
    return pl.pallas_call(
        _proj_body,
        grid=(V, NC_H),
        in_specs=[
            pl.BlockSpec((1, 4, 4), lambda v, c: (v, 0, 0)),
            pl.BlockSpec((3, 3), lambda v, c: (0, 0)),
            pl.BlockSpec((1, 4, CH), lambda v, c: (v, 0, c)),
            pix(), pix(),
            full(), full(),
        ],
        out_specs=[pix(), pix(), full()],
        out_shape=[
            jax.ShapeDtypeStruct((V, NC_H, 1, CH), jnp.int32),
            jax.ShapeDtypeStruct((V, NC_H, 1, CH), jnp.float32),
            jax.ShapeDtypeStruct((H, W), jnp.float32),
        ],
    )(Ts, K, hmat, cls4, conf4, maskf, depths)


def _gather_compare(lin, dm, table):
    info = plsc.get_sparse_core_info()
    nc, ns = info.num_cores, info.num_subcores
    nw = nc * ns
    per_w = NT // nw
    nchunk = per_w // C
    mesh = plsc.VectorSubcoreMesh(core_axis_name="c", subcore_axis_name="s")

    @functools.partial(
        pl.kernel,
        mesh=mesh,
        out_type=jax.ShapeDtypeStruct((NT,), jnp.int32),
        scratch_types=[
            pltpu.VMEM_SHARED((N,), jnp.float32),
            pltpu.VMEM((C,), jnp.int32),
            pltpu.VMEM((C,), jnp.int32),
            pltpu.VMEM((2, C), jnp.float32),
            pltpu.VMEM((C,), jnp.float32),
            pltpu.VMEM((2, C), jnp.int32),
            pltpu.SemaphoreType.DMA,
            pltpu.SemaphoreType.DMA,
            pltpu.SemaphoreType.DMA,
        ],
    )
    def k(lin_hbm, dm_hbm, tab_hbm, out_hbm, tab_s, idx0, idx1, d_v, g_v,
          o_v, sin, sg, so):
        sid = lax.axis_index("s")
        wid = sid * nc + lax.axis_index("c")
        base = wid * per_w

        @pl.when(sid == 0)
        def _():
            pltpu.sync_copy(tab_hbm, tab_s)

        plsc.subcore_barrier()

        idxs = (idx0, idx1)

        def start_in(j):
            off = base + j * C
            pltpu.async_copy(lin_hbm.at[pl.ds(off, C)], idxs[j % 2], sin)
            pltpu.async_copy(dm_hbm.at[pl.ds(off, C)], d_v.at[j % 2], sin)

        start_in(0)
        for j in range(nchunk):
            slot = j % 2
            if j + 1 < nchunk:
                start_in(j + 1)
            pltpu.make_async_copy(lin_hbm.at[pl.ds(base, C)],
                                  idxs[slot], sin).wait()
            pltpu.make_async_copy(dm_hbm.at[pl.ds(base, C)],
                                  d_v.at[slot], sin).wait()
            pltpu.async_copy(tab_s.at[idxs[slot]], g_v, sg).wait()

            if j >= 2:
                pltpu.make_async_copy(o_v.at[slot],
                                      out_hbm.at[pl.ds(base, C)], so).wait()

            def cmp16(i, _, slot=slot):
                sl = pl.ds(i * 16, 16)
                cmp = d_v[slot, sl] > g_v[sl]
                o_v[slot, sl] = lax.select(cmp, jnp.ones((16,), jnp.int32),
                                           jnp.zeros((16,), jnp.int32))
                return 0

            lax.fori_loop(0, C // 16, cmp16, 0, unroll=8)
            pltpu.async_copy(o_v.at[slot], out_hbm.at[pl.ds(base + j * C, C)],
                             so)
        for _ in range(min(nchunk, 2)):
            pltpu.make_async_copy(o_v.at[0], out_hbm.at[pl.ds(base, C)],
                                  so).wait()

    return k(lin, dm, table)


CP = 4096


def _deinterleave(pm_flat):
    info = plsc.get_sparse_core_info()
    nc, ns = info.num_cores, info.num_subcores
    nw = nc * ns
    pix_w = V * N // nw
    nchunk = pix_w // CP
    qpv = nw // V
    mesh = plsc.VectorSubcoreMesh(core_axis_name="c", subcore_axis_name="s")

    @functools.partial(
        pl.kernel,
        mesh=mesh,
        out_type=jax.ShapeDtypeStruct((V * 4 * N,), jnp.float32),
        scratch_types=[
            pltpu.VMEM((CP,), jnp.int32),
            pltpu.VMEM((CP,), jnp.int32),
            pltpu.VMEM((CP,), jnp.int32),
            pltpu.VMEM((CP,), jnp.float32),
            pltpu.VMEM((CP,), jnp.float32),
            pltpu.VMEM((CP,), jnp.float32),
            pltpu.VMEM((CP,), jnp.float32),
            pltpu.SemaphoreType.DMA,
            pltpu.SemaphoreType.DMA,
        ],
    )
    def k(pm_hbm, h_hbm, ixb, iyb, izb, bx, by, bz, bones, sg, so):
        wid = lax.axis_index("s") * nc + lax.axis_index("c")
        v = wid // qpv
        pix0 = (wid % qpv) * pix_w

        def fill16(i, _):
            sl = pl.ds(i * 16, 16)
            base16 = lax.iota(jnp.int32, 16) * 3 + i * 48
            ixb[sl] = base16
            iyb[sl] = base16 + 1
            izb[sl] = base16 + 2
            bones[sl] = jnp.full((16,), 1.0, jnp.float32)
            return 0

        lax.fori_loop(0, CP // 16, fill16, 0, unroll=8)

        for j in range(nchunk):
            src = pm_hbm.at[pl.ds((v * N + pix0 + j * CP) * 3, 3 * CP)]
            pltpu.async_copy(src.at[ixb], bx, sg)
            pltpu.async_copy(src.at[iyb], by, sg)
            pltpu.async_copy(src.at[izb], bz, sg)
            if j > 0:
                for _ in range(3):
                    pltpu.make_async_copy(bones, h_hbm.at[pl.ds(0, CP)],
                                          so).wait()
            for _ in range(3):
                pltpu.make_async_copy(h_hbm.at[pl.ds(0, CP)], bx, sg).wait()
            off = pix0 + j * CP
            pltpu.async_copy(bx, h_hbm.at[pl.ds((v * 4 + 0) * N + off, CP)], so)
            pltpu.async_copy(by, h_hbm.at[pl.ds((v * 4 + 1) * N + off, CP)], so)
            pltpu.async_copy(bz, h_hbm.at[pl.ds((v * 4 + 2) * N + off, CP)], so)
            pltpu.sync_copy(bones, h_hbm.at[pl.ds((v * 4 + 3) * N + off, CP)])
        for _ in range(3):
            pltpu.make_async_copy(bones, h_hbm.at[pl.ds(0, CP)], so).wait()

    return k(pm_flat)


def kernel(pointmaps, classifier, conf_pointmaps, input_mask, input_c2w,
           input_K, input_depths, new_c2ws):
    input_w2c = jnp.linalg.inv(input_c2w)
    Ts = jax.vmap(lambda c2w: input_w2c @ c2w)(new_c2ws)

    hmat = _deinterleave(pointmaps.reshape(-1)).reshape(V, 4, N)
    cls4 = classifier.reshape(V, NC_H, 1, CH)
    conf4 = conf_pointmaps.reshape(V, NC_H, 1, CH)
    maskf = input_mask.astype(jnp.float32)

    lin, dm, table = _project(Ts, input_K, hmat, cls4, conf4, maskf,
                              input_depths)
    out = _gather_compare(lin.reshape(-1), dm.reshape(-1), table.reshape(-1))
    return out.astype(bool).reshape(1, V, H, W)


# speedup vs baseline: 1.0654x; 1.0654x over previous
import functools

import jax
import jax.numpy as jnp
from jax import lax
from jax.experimental import pallas as pl
from jax.experimental.pallas import tpu as pltpu
from jax.experimental.pallas import tpu_sc as plsc

V, H, W = 8, 512, 512
N = H * W
NT = V * N
CH = 16384
NC_H = N // CH
C = 8192


def _proj_body(T_ref, K_ref, h_ref, cls_ref, conf_ref, maskf_ref, dep_ref,
               lin_ref, dm_ref, tab_ref):
    v = pl.program_id(0)
    c = pl.program_id(1)

    hT = h_ref[0]
    T = T_ref[0]
    wh = jax.lax.dot_general(T, hT, (((1,), (0,)), ((), ())),
                             preferred_element_type=jnp.float32)
    pc = wh[0:3] / wh[3:4]
    img = jax.lax.dot_general(K_ref[...], pc, (((1,), (0,)), ((), ())),
                              preferred_element_type=jnp.float32)
    uv = img[0:2] / img[2:3]
    ij = uv.astype(jnp.int32)
    ix = ij[0:1]
    iy = ij[1:2]
    valid = (ix >= 0) & (ix < W) & (iy >= 0) & (iy < H)
    ixc = jnp.clip(ix, 0, W - 1)
    iyc = jnp.clip(iy, 0, H - 1)

    keep = valid & (cls_ref[0, 0] > 0.0) & (conf_ref[0, 0] > 0.5)
    spread = jax.lax.broadcasted_iota(jnp.int32, (1, CH), 1) & (N - 1)
    lin_ref[0, 0] = jnp.where(keep, iyc * W + ixc, spread)
    dm_ref[0, 0] = jnp.where(keep, pc[2:3], -jnp.inf)

    @pl.when((v == 0) & (c == 0))
    def _():
        tab_ref[...] = jnp.where(maskf_ref[...] > 0.0, dep_ref[...], jnp.inf)


def _project(Ts, K, hmat, cls4, conf4, maskf, depths):
    full = lambda: pl.BlockSpec((H, W), lambda v, c: (0, 0))
    pix = lambda: pl.BlockSpec((1, 1, 1, CH), lambda v, c: (v, c, 0, 0))
    return pl.pallas_call(
        _proj_body,
        grid=(V, NC_H),
        in_specs=[
            pl.BlockSpec((1, 4, 4), lambda v, c: (v, 0, 0)),
            pl.BlockSpec((3, 3), lambda v, c: (0, 0)),
            pl.BlockSpec((1, 4, CH), lambda v, c: (v, 0, c)),
            pix(), pix(),
            full(), full(),
        ],
        out_specs=[pix(), pix(), full()],
        out_shape=[
            jax.ShapeDtypeStruct((V, NC_H, 1, CH), jnp.int32),
            jax.ShapeDtypeStruct((V, NC_H, 1, CH), jnp.float32),
            jax.ShapeDtypeStruct((H, W), jnp.float32),
        ],
    )(Ts, K, hmat, cls4, conf4, maskf, depths)


def _gather_compare(lin, dm, table):
    info = plsc.get_sparse_core_info()
    nc, ns = info.num_cores, info.num_subcores
    nw = nc * ns
    per_w = NT // nw
    nchunk = per_w // C
    mesh = plsc.VectorSubcoreMesh(core_axis_name="c", subcore_axis_name="s")

    @functools.partial(
        pl.kernel,
        mesh=mesh,
        out_type=jax.ShapeDtypeStruct((NT,), jnp.int32),
        scratch_types=[
            pltpu.VMEM_SHARED((N,), jnp.float32),
            pltpu.VMEM((C,), jnp.int32),
            pltpu.VMEM((C,), jnp.int32),
            pltpu.VMEM((2, C), jnp.float32),
            pltpu.VMEM((C,), jnp.float32),
            pltpu.VMEM((2, C), jnp.int32),
            pltpu.SemaphoreType.DMA,
            pltpu.SemaphoreType.DMA,
            pltpu.SemaphoreType.DMA,
        ],
    )
    def k(lin_hbm, dm_hbm, tab_hbm, out_hbm, tab_s, idx0, idx1, d_v, g_v,
          o_v, sin, sg, so):
        sid = lax.axis_index("s")
        wid = sid * nc + lax.axis_index("c")
        base = wid * per_w

        @pl.when(sid == 0)
        def _():
            pltpu.sync_copy(tab_hbm, tab_s)

        plsc.subcore_barrier()

        idxs = (idx0, idx1)

        def start_in(j):
            off = base + j * C
            pltpu.async_copy(lin_hbm.at[pl.ds(off, C)], idxs[j % 2], sin)
            pltpu.async_copy(dm_hbm.at[pl.ds(off, C)], d_v.at[j % 2], sin)

        start_in(0)
        for j in range(nchunk):
            slot = j % 2
            if j + 1 < nchunk:
                start_in(j + 1)
            pltpu.make_async_copy(lin_hbm.at[pl.ds(base, C)],
                                  idxs[slot], sin).wait()
            pltpu.make_async_copy(dm_hbm.at[pl.ds(base, C)],
                                  d_v.at[slot], sin).wait()
            pltpu.async_copy(tab_s.at[idxs[slot]], g_v, sg).wait()

            if j >= 2:
                pltpu.make_async_copy(o_v.at[slot],
                                      out_hbm.at[pl.ds(base, C)], so).wait()

            def cmp16(i, _, slot=slot):
                sl = pl.ds(i * 16, 16)
                cmp = d_v[slot, sl] > g_v[sl]
                o_v[slot, sl] = lax.select(cmp, jnp.ones((16,), jnp.int32),
                                           jnp.zeros((16,), jnp.int32))
                return 0

            lax.fori_loop(0, C // 16, cmp16, 0, unroll=8)
            pltpu.async_copy(o_v.at[slot], out_hbm.at[pl.ds(base + j * C, C)],
                             so)
        for _ in range(min(nchunk, 2)):
            pltpu.make_async_copy(o_v.at[0], out_hbm.at[pl.ds(base, C)],
                                  so).wait()

    return k(lin, dm, table)


CP = 4096


def _deinterleave(pm_flat):
    info = plsc.get_sparse_core_info()
    nc, ns = info.num_cores, info.num_subcores
    nw = nc * ns
    pix_w = V * N // nw
    nchunk = pix_w // CP
    qpv = nw // V
    mesh = plsc.VectorSubcoreMesh(core_axis_name="c", subcore_axis_name="s")

    @functools.partial(
        pl.kernel,
        mesh=mesh,
        out_type=jax.ShapeDtypeStruct((V * 4 * N,), jnp.float32),
        scratch_types=[
            pltpu.VMEM((CP,), jnp.int32),
            pltpu.VMEM((CP,), jnp.int32),
            pltpu.VMEM((CP,), jnp.int32),
            pltpu.VMEM((CP,), jnp.float32),
            pltpu.VMEM((CP,), jnp.float32),
            pltpu.VMEM((CP,), jnp.float32),
            pltpu.VMEM((CP,), jnp.float32),
            pltpu.VMEM_SHARED((2 * 16 * 3 * CP,), jnp.float32),
            pltpu.SemaphoreType.DMA,
            pltpu.SemaphoreType.DMA,
            pltpu.SemaphoreType.DMA,
        ],
    )
    def k(pm_hbm, h_hbm, ixb, iyb, izb, bx, by, bz, bones, spm, ss, sg, so):
        sid = lax.axis_index("s")
        wid = sid * nc + lax.axis_index("c")
        v = wid // qpv
        pix0 = (wid % qpv) * pix_w

        def fill16(i, _):
            sl = pl.ds(i * 16, 16)
            base16 = lax.iota(jnp.int32, 16) * 3 + i * 48
            ixb[sl] = base16
            iyb[sl] = base16 + 1
            izb[sl] = base16 + 2
            bones[sl] = jnp.full((16,), 1.0, jnp.float32)
            return 0

        lax.fori_loop(0, CP // 16, fill16, 0, unroll=8)

        def stage(j):
            off_h = (v * N + pix0 + j * CP) * 3
            off_s = ((j % 2) * 16 + sid) * (3 * CP)
            pltpu.async_copy(pm_hbm.at[pl.ds(off_h, 3 * CP)],
                             spm.at[pl.ds(off_s, 3 * CP)], ss)

        stage(0)
        for j in range(nchunk):
            if j + 1 < nchunk:
                stage(j + 1)
            pltpu.make_async_copy(pm_hbm.at[pl.ds(0, 3 * CP)],
                                  spm.at[pl.ds(0, 3 * CP)], ss).wait()
            src = spm.at[pl.ds(((j % 2) * 16 + sid) * (3 * CP), 3 * CP)]
            pltpu.async_copy(src.at[ixb], bx, sg)
            pltpu.async_copy(src.at[iyb], by, sg)
            pltpu.async_copy(src.at[izb], bz, sg)
            if j > 0:
                for _ in range(3):
                    pltpu.make_async_copy(bones, h_hbm.at[pl.ds(0, CP)],
                                          so).wait()
            for _ in range(3):
                pltpu.make_async_copy(h_hbm.at[pl.ds(0, CP)], bx, sg).wait()
            off = pix0 + j * CP
            pltpu.async_copy(bx, h_hbm.at[pl.ds((v * 4 + 0) * N + off, CP)], so)
            pltpu.async_copy(by, h_hbm.at[pl.ds((v * 4 + 1) * N + off, CP)], so)
            pltpu.async_copy(bz, h_hbm.at[pl.ds((v * 4 + 2) * N + off, CP)], so)
            pltpu.sync_copy(bones, h_hbm.at[pl.ds((v * 4 + 3) * N + off, CP)])
        for _ in range(3):
            pltpu.make_async_copy(bones, h_hbm.at[pl.ds(0, CP)], so).wait()

    return k(pm_flat)


def kernel(pointmaps, classifier, conf_pointmaps, input_mask, input_c2w,
           input_K, input_depths, new_c2ws):
    input_w2c = jnp.linalg.inv(input_c2w)
    Ts = jax.vmap(lambda c2w: input_w2c @ c2w)(new_c2ws)

    hmat = _deinterleave(pointmaps.reshape(-1)).reshape(V, 4, N)
    cls4 = classifier.reshape(V, NC_H, 1, CH)
    conf4 = conf_pointmaps.reshape(V, NC_H, 1, CH)
    maskf = input_mask.astype(jnp.float32)

    lin, dm, table = _project(Ts, input_K, hmat, cls4, conf4, maskf,
                              input_depths)
    out = _gather_compare(lin.reshape(-1), dm.reshape(-1), table.reshape(-1))
    return out.astype(bool).reshape(1, V, H, W)


# speedup vs baseline: 9.4005x; 8.8234x over previous
import functools

import jax
import jax.numpy as jnp
from jax import lax
from jax.experimental import pallas as pl
from jax.experimental.pallas import tpu as pltpu
from jax.experimental.pallas import tpu_sc as plsc

V, H, W = 8, 512, 512
N = H * W
NT = V * N
CH = 16384
NC_H = N // CH
C = 8192


def _proj_body(T_ref, K_ref, h_ref, cls_ref, conf_ref, maskf_ref, dep_ref,
               lin_ref, dm_ref, tab_ref):
    v = pl.program_id(0)
    c = pl.program_id(1)

    hT = h_ref[0]
    T = T_ref[0]
    wh = jax.lax.dot_general(T, hT, (((1,), (0,)), ((), ())),
                             preferred_element_type=jnp.float32)
    pc = wh[0:3] / wh[3:4]
    img = jax.lax.dot_general(K_ref[...], pc, (((1,), (0,)), ((), ())),
                              preferred_element_type=jnp.float32)
    uv = img[0:2] / img[2:3]
    ij = uv.astype(jnp.int32)
    ix = ij[0:1]
    iy = ij[1:2]
    valid = (ix >= 0) & (ix < W) & (iy >= 0) & (iy < H)
    ixc = jnp.clip(ix, 0, W - 1)
    iyc = jnp.clip(iy, 0, H - 1)

    keep = valid & (cls_ref[...][None] > 0.0) & (conf_ref[...][None] > 0.5)
    spread = jax.lax.broadcasted_iota(jnp.int32, (1, CH), 1) & (N - 1)
    lin_ref[...] = jnp.where(keep, iyc * W + ixc, spread)[0]
    dm_ref[...] = jnp.where(keep, pc[2:3], -jnp.inf)[0]

    @pl.when((v == 0) & (c == 0))
    def _():
        tab_ref[...] = jnp.where(maskf_ref[...] > 0.0, dep_ref[...], jnp.inf)


def _project(Ts, K, hmat, cls4, conf4, maskf, depths):
    full = lambda: pl.BlockSpec((H, W), lambda v, c: (0, 0))
    pix = lambda: pl.BlockSpec((CH,), lambda v, c: (v * NC_H + c,))
    return pl.pallas_call(
        _proj_body,
        grid=(V, NC_H),
        in_specs=[
            pl.BlockSpec((1, 4, 4), lambda v, c: (v, 0, 0)),
            pl.BlockSpec((3, 3), lambda v, c: (0, 0)),
            pl.BlockSpec((1, 4, CH), lambda v, c: (v, 0, c)),
            pix(), pix(),
            full(), full(),
        ],
        out_specs=[pix(), pix(), full()],
        out_shape=[
            jax.ShapeDtypeStruct((NT,), jnp.int32),
            jax.ShapeDtypeStruct((NT,), jnp.float32),
            jax.ShapeDtypeStruct((H, W), jnp.float32),
        ],
    )(Ts, K, hmat, cls4, conf4, maskf, depths)


def _gather_compare(lin, dm, table):
    info = plsc.get_sparse_core_info()
    nc, ns = info.num_cores, info.num_subcores
    nw = nc * ns
    per_w = NT // nw
    nchunk = per_w // C
    mesh = plsc.VectorSubcoreMesh(core_axis_name="c", subcore_axis_name="s")

    @functools.partial(
        pl.kernel,
        mesh=mesh,
        out_type=jax.ShapeDtypeStruct((NT,), jnp.int32),
        scratch_types=[
            pltpu.VMEM_SHARED((N,), jnp.float32),
            pltpu.VMEM((C,), jnp.int32),
            pltpu.VMEM((C,), jnp.int32),
            pltpu.VMEM((2, C), jnp.float32),
            pltpu.VMEM((C,), jnp.float32),
            pltpu.VMEM((2, C), jnp.int32),
            pltpu.SemaphoreType.DMA,
            pltpu.SemaphoreType.DMA,
            pltpu.SemaphoreType.DMA,
        ],
    )
    def k(lin_hbm, dm_hbm, tab_hbm, out_hbm, tab_s, idx0, idx1, d_v, g_v,
          o_v, sin, sg, so):
        sid = lax.axis_index("s")
        wid = sid * nc + lax.axis_index("c")
        base = wid * per_w

        @pl.when(sid == 0)
        def _():
            pltpu.sync_copy(tab_hbm, tab_s)

        plsc.subcore_barrier()

        idxs = (idx0, idx1)

        def start_in(j):
            off = base + j * C
            pltpu.async_copy(lin_hbm.at[pl.ds(off, C)], idxs[j % 2], sin)
            pltpu.async_copy(dm_hbm.at[pl.ds(off, C)], d_v.at[j % 2], sin)

        start_in(0)
        for j in range(nchunk):
            slot = j % 2
            if j + 1 < nchunk:
                start_in(j + 1)
            pltpu.make_async_copy(lin_hbm.at[pl.ds(base, C)],
                                  idxs[slot], sin).wait()
            pltpu.make_async_copy(dm_hbm.at[pl.ds(base, C)],
                                  d_v.at[slot], sin).wait()
            pltpu.async_copy(tab_s.at[idxs[slot]], g_v, sg).wait()

            if j >= 2:
                pltpu.make_async_copy(o_v.at[slot],
                                      out_hbm.at[pl.ds(base, C)], so).wait()

            def cmp16(i, _, slot=slot):
                sl = pl.ds(i * 16, 16)
                cmp = d_v[slot, sl] > g_v[sl]
                o_v[slot, sl] = lax.select(cmp, jnp.ones((16,), jnp.int32),
                                           jnp.zeros((16,), jnp.int32))
                return 0

            lax.fori_loop(0, C // 16, cmp16, 0, unroll=8)
            pltpu.async_copy(o_v.at[slot], out_hbm.at[pl.ds(base + j * C, C)],
                             so)
        for _ in range(min(nchunk, 2)):
            pltpu.make_async_copy(o_v.at[0], out_hbm.at[pl.ds(base, C)],
                                  so).wait()

    return k(lin, dm, table)


def kernel(pointmaps, classifier, conf_pointmaps, input_mask, input_c2w,
           input_K, input_depths, new_c2ws):
    input_w2c = jnp.linalg.inv(input_c2w)
    Ts = jax.vmap(lambda c2w: input_w2c @ c2w)(new_c2ws)

    hmat = jnp.stack([pointmaps[..., 0].reshape(V, N),
                      pointmaps[..., 1].reshape(V, N),
                      pointmaps[..., 2].reshape(V, N),
                      jnp.ones((V, N), jnp.float32)], axis=1)
    cls4 = classifier.reshape(NT)
    conf4 = conf_pointmaps.reshape(NT)
    maskf = input_mask.astype(jnp.float32)

    lin, dm, table = _project(Ts, input_K, hmat, cls4, conf4, maskf,
                              input_depths)
    out = _gather_compare(lin, dm, table.reshape(-1))
    return out.astype(bool).reshape(1, V, H, W)
